# Initial kernel scaffold; baseline (speedup 1.0000x reference)
import jax
import jax.numpy as jnp
from jax.experimental import pallas as pl


def kernel(x, edge_index, edge_type, indice_pairs, W1, W1_root, W2, W2_root):
    raise NotImplementedError("write your pallas kernel here")


# speedup vs baseline: 21.9797x; 21.9797x over previous
import functools
import jax
import jax.numpy as jnp
from jax import lax
from jax.experimental import pallas as pl
from jax.experimental.pallas import tpu as pltpu
from jax.experimental.pallas import tpu_sc as plsc

N = 10000
E = 320000
NR = 8
D = 128
NK = NR * N
ACC_ROWS = 90112
E_PAD = 327680
EROWS = E_PAD // 128
ROWS_PER_TILE = EROWS // 16
NC, NS = 2, 16

_mesh = plsc.VectorSubcoreMesh(
    core_axis_name="c", subcore_axis_name="s", num_cores=NC, num_subcores=NS)


def _zero_zbuf(zbuf):
    def zb(i, carry):
        zbuf[i, :] = jnp.zeros((16,), jnp.float32)
        return carry
    lax.fori_loop(0, zbuf.shape[0], zb, 0)


@functools.partial(
    pl.kernel,
    out_type=[
        jax.ShapeDtypeStruct((EROWS, 128), jnp.int32),
        jax.ShapeDtypeStruct((32, 81920), jnp.float32),
    ],
    mesh=_mesh,
    scratch_types=[
        pltpu.VMEM((16, 128), jnp.int32),
        pltpu.VMEM((16, 128), jnp.int32),
        pltpu.VMEM((16, 128), jnp.int32),
        pltpu.VMEM((81920,), jnp.float32),
    ],
    compiler_params=pltpu.CompilerParams(needs_layout_passes=False),
)
def _deg_kernel(dst_rows, ty_rows, kidx_rows, degp, dbuf, tbuf, kbuf, degacc):
    sc = lax.axis_index("c")
    s = lax.axis_index("s")
    w = sc * NS + s

    def zacc(i, carry):
        degacc[pl.ds(i * 16, 16)] = jnp.zeros((16,), jnp.float32)
        return carry
    lax.fori_loop(0, 81920 // 16, zacc, 0)

    ones = jnp.ones((16,), jnp.float32)
    row0 = w * (EROWS // 32)
    for g in range(EROWS // 32 // 16):
        g0 = row0 + g * 16
        pltpu.sync_copy(dst_rows.at[pl.ds(g0, 16)], dbuf)
        pltpu.sync_copy(ty_rows.at[pl.ds(g0, 16)], tbuf)

        def rowfn(r, carry):
            for i in range(8):
                d = dbuf[r, pl.ds(i * 16, 16)]
                t = tbuf[r, pl.ds(i * 16, 16)]
                kbuf[r, pl.ds(i * 16, 16)] = t * N + d
                kd = d * NR + t
                plsc.addupdate_scatter(degacc, [kd], ones)
            return carry
        lax.fori_loop(0, 16, rowfn, 0)
        pltpu.sync_copy(kbuf, kidx_rows.at[pl.ds(g0, 16)])
    pltpu.sync_copy(degacc, degp.at[w])


@functools.partial(
    pl.kernel,
    out_type=[
        jax.ShapeDtypeStruct((NK, D), jnp.float32),
        jax.ShapeDtypeStruct((NR, N, 16), jnp.float32),
    ],
    mesh=_mesh,
    scratch_types=[
        pltpu.VMEM((625, 16), jnp.float32),
        pltpu.VMEM((128, 16), jnp.float32),
        pltpu.VMEM((8, 128), jnp.int32),
        pltpu.VMEM((8, 128), jnp.int32),
        pltpu.VMEM((1024, 16), jnp.float32),
        pltpu.VMEM_SHARED((ACC_ROWS, 16), jnp.float32),
        pltpu.SemaphoreType.DMA,
        pltpu.SemaphoreType.DMA,
    ],
    compiler_params=pltpu.CompilerParams(use_tc_tiling_on_sc=False),
)
def _agg_kernel(table, src_rows, kidx_rows, agg, tab_ch, relay, zbuf, srcb,
                kb, vals, acc, semg, sems):
    sc = lax.axis_index("c")
    s = lax.axis_index("s")
    cl = s // 4
    q = s % 4
    c = sc * 4 + cl
    for i in range(4):
        r0 = q * 2500 + i * 625
        pltpu.sync_copy(table.at[pl.ds(r0, 625), pl.ds(c * 16, 16)], relay)
        pltpu.sync_copy(relay, tab_ch.at[c, pl.ds(r0, 625), :])
    _zero_zbuf(zbuf)
    plsc.subcore_barrier()

    for cl2 in range(4):
        c2 = sc * 4 + cl2

        def zacc(i, carry):
            pltpu.sync_copy(zbuf, acc.at[pl.ds(s * 5632 + i * 128, 128)])
            return carry
        lax.fori_loop(0, 44, zacc, 0)
        plsc.subcore_barrier()

        def batch(b, carry):
            r0 = s * ROWS_PER_TILE + b * 8
            pltpu.sync_copy(src_rows.at[pl.ds(r0, 8)], srcb)
            pltpu.sync_copy(kidx_rows.at[pl.ds(r0, 8)], kb)
            gathers = [
                pltpu.async_copy(tab_ch.at[c2].at[srcb.at[j]],
                                 vals.at[pl.ds(j * 128, 128)], semg)
                for j in range(8)
            ]
            for g in gathers:
                g.wait()
            scatters = [
                pltpu.async_copy(vals.at[pl.ds(j * 128, 128)],
                                 acc.at[kb.at[j]], sems, add=True)
                for j in range(8)
            ]
            for sct in scatters:
                sct.wait()
            return carry
        lax.fori_loop(0, ROWS_PER_TILE // 8, batch, 0)
        plsc.subcore_barrier()
        pltpu.sync_copy(acc.at[pl.ds(s * 5000, 5000)],
                        agg.at[pl.ds(s * 5000, 5000), pl.ds(c2 * 16, 16)])
        plsc.subcore_barrier()


@functools.partial(
    pl.kernel,
    out_type=jax.ShapeDtypeStruct((8192, D), jnp.float32),
    mesh=_mesh,
    scratch_types=[
        pltpu.VMEM((2, 128), jnp.int32),
        pltpu.VMEM((256, D), jnp.float32),
        pltpu.SemaphoreType.DMA,
    ],
    compiler_params=pltpu.CompilerParams(use_tc_tiling_on_sc=False),
)
def _pair_kernel(z, pair_rows, out, ib, vals, sem):
    w = lax.axis_index("c") * NS + lax.axis_index("s")
    pltpu.sync_copy(pair_rows.at[pl.ds(w * 2, 2)], ib)
    copies = [
        pltpu.async_copy(z.at[ib.at[r]], vals.at[pl.ds(r * 128, 128)], sem)
        for r in range(2)
    ]
    for cp in copies:
        cp.wait()
    pltpu.sync_copy(vals, out.at[pl.ds(w * 256, 256)])


def _mm_body(x_ref, agg_ref, degp_ref, wf_ref, wr_ref, o_ref, *, relu):
    acc = jnp.dot(x_ref[...], wr_ref[...],
                  preferred_element_type=jnp.float32,
                  precision=lax.Precision.HIGHEST)
    deg = jnp.sum(degp_ref[...], axis=0)
    inv = 1.0 / jnp.maximum(deg, 1.0)
    for r in range(NR):
        acc = acc + jnp.dot(agg_ref[r] * inv[:, r:r + 1], wf_ref[r],
                            preferred_element_type=jnp.float32,
                            precision=lax.Precision.HIGHEST)
    o_ref[...] = jnp.maximum(acc, 0.0) if relu else acc


def _mm_call(xin, agg3, degp3, wf, wr, relu):
    blk = 400
    do = wf.shape[2]
    return pl.pallas_call(
        functools.partial(_mm_body, relu=relu),
        grid=(N // blk,),
        in_specs=[
            pl.BlockSpec((blk, D), lambda i: (i, 0)),
            pl.BlockSpec((NR, blk, D), lambda i: (0, i, 0)),
            pl.BlockSpec((32, blk, NR), lambda i: (0, i, 0)),
            pl.BlockSpec((NR, D, do), lambda i: (0, 0, 0)),
            pl.BlockSpec((D, do), lambda i: (0, 0)),
        ],
        out_specs=pl.BlockSpec((blk, do), lambda i: (i, 0)),
        out_shape=jax.ShapeDtypeStruct((N, do), jnp.float32),
    )(xin, agg3, degp3, wf, wr)


def kernel(x, edge_index, edge_type, indice_pairs, W1, W1_root, W2, W2_root):
    src = edge_index[0]
    dst = edge_index[1]
    pad = E_PAD - E
    src_p = jnp.concatenate([src, jnp.zeros((pad,), jnp.int32)]).reshape(EROWS, 128)
    dst_p = jnp.concatenate([dst, jnp.full((pad,), N, jnp.int32)]).reshape(EROWS, 128)
    ty_p = jnp.concatenate([edge_type, jnp.full((pad,), NR, jnp.int32)]).reshape(EROWS, 128)

    kidx_rows, degp = _deg_kernel(dst_p, ty_p)
    degp3 = degp[:, :NK].reshape(32, N, NR)

    agg1, _ = _agg_kernel(x, src_p, kidx_rows)
    h = _mm_call(x, agg1.reshape(NR, N, D), degp3, W1, W1_root, relu=True)

    agg2, _ = _agg_kernel(h, src_p, kidx_rows)
    w2f = jnp.concatenate([W2, jnp.zeros((NR, D, 78), jnp.float32)], axis=2)
    w2r = jnp.concatenate([W2_root, jnp.zeros((D, 78), jnp.float32)], axis=1)
    z = _mm_call(h, agg2.reshape(NR, N, D), degp3, w2f, w2r, relu=False)

    pair_rows = jnp.concatenate(
        [indice_pairs[:, 0], indice_pairs[:, 1]]).reshape(64, 128)
    zz = _pair_kernel(z, pair_rows)
    return (zz[:4096, :50], zz[4096:, :50])


# speedup vs baseline: 25.2953x; 1.1509x over previous
import functools
import jax
import jax.numpy as jnp
from jax import lax
from jax.experimental import pallas as pl
from jax.experimental.pallas import tpu as pltpu
from jax.experimental.pallas import tpu_sc as plsc

N = 10000
E = 320000
NR = 8
D = 128
NK = NR * N
ACC_ROWS = 81920
E_PAD = 327680
EROWS = E_PAD // 128
ROWS_PER_TILE = EROWS // 16
NC, NS = 2, 16

_mesh = plsc.VectorSubcoreMesh(
    core_axis_name="c", subcore_axis_name="s", num_cores=NC, num_subcores=NS)


def _zero_zbuf(zbuf):
    def zb(i, carry):
        zbuf[i, :] = jnp.zeros((16,), jnp.float32)
        return carry
    lax.fori_loop(0, zbuf.shape[0], zb, 0)


@functools.partial(
    pl.kernel,
    out_type=[
        jax.ShapeDtypeStruct((EROWS, 128), jnp.int32),
        jax.ShapeDtypeStruct((32, 81920), jnp.float32),
    ],
    mesh=_mesh,
    scratch_types=[
        pltpu.VMEM((16, 128), jnp.int32),
        pltpu.VMEM((16, 128), jnp.int32),
        pltpu.VMEM((16, 128), jnp.int32),
        pltpu.VMEM((81920,), jnp.float32),
    ],
    compiler_params=pltpu.CompilerParams(needs_layout_passes=False),
)
def _deg_kernel(dst_rows, ty_rows, kidx_rows, degp, dbuf, tbuf, kbuf, degacc):
    sc = lax.axis_index("c")
    s = lax.axis_index("s")
    w = sc * NS + s

    def zacc(i, carry):
        degacc[pl.ds(i * 16, 16)] = jnp.zeros((16,), jnp.float32)
        return carry
    lax.fori_loop(0, 81920 // 16, zacc, 0)

    ones = jnp.ones((16,), jnp.float32)
    row0 = w * (EROWS // 32)
    for g in range(EROWS // 32 // 16):
        g0 = row0 + g * 16
        pltpu.sync_copy(dst_rows.at[pl.ds(g0, 16)], dbuf)
        pltpu.sync_copy(ty_rows.at[pl.ds(g0, 16)], tbuf)

        def rowfn(r, carry):
            for i in range(8):
                d = dbuf[r, pl.ds(i * 16, 16)]
                t = tbuf[r, pl.ds(i * 16, 16)]
                kbuf[r, pl.ds(i * 16, 16)] = t * N + d
                kd = jnp.where(t >= NR, 80008, d * NR + t)
                plsc.addupdate_scatter(degacc, [kd], ones)
            return carry
        lax.fori_loop(0, 16, rowfn, 0)
        pltpu.sync_copy(kbuf, kidx_rows.at[pl.ds(g0, 16)])
    pltpu.sync_copy(degacc, degp.at[w])


@functools.partial(
    pl.kernel,
    out_type=[
        jax.ShapeDtypeStruct((NK, D), jnp.float32),
        jax.ShapeDtypeStruct((NR, N, 16), jnp.float32),
    ],
    mesh=_mesh,
    scratch_types=[
        pltpu.VMEM((250, 16), jnp.float32),
        pltpu.VMEM((128, 16), jnp.float32),
        pltpu.VMEM((2, 8, 128), jnp.int32),
        pltpu.VMEM((2, 8, 128), jnp.int32),
        pltpu.VMEM((2, 1024, 16), jnp.float32),
        pltpu.VMEM_SHARED((ACC_ROWS, 16), jnp.float32),
        pltpu.SemaphoreType.DMA,
        pltpu.SemaphoreType.DMA,
    ],
    compiler_params=pltpu.CompilerParams(use_tc_tiling_on_sc=False),
)
def _agg_kernel(table, src_rows, kidx_rows, agg, tab_ch, relay, zbuf, srcb,
                kb, vals, acc, semg, sems):
    sc = lax.axis_index("c")
    s = lax.axis_index("s")
    nb = ROWS_PER_TILE // 8
    cl = s // 4
    q = s % 4
    c = sc * 4 + cl
    for i in range(10):
        r0 = q * 2500 + i * 250
        pltpu.sync_copy(table.at[pl.ds(r0, 250), pl.ds(c * 16, 16)], relay)
        pltpu.sync_copy(relay, tab_ch.at[c, pl.ds(r0, 250), :])
    _zero_zbuf(zbuf)
    plsc.subcore_barrier()

    def load_idx(gb, buf):
        r0 = s * ROWS_PER_TILE + gb * 8
        pltpu.sync_copy(src_rows.at[pl.ds(r0, 8)], srcb.at[buf])
        pltpu.sync_copy(kidx_rows.at[pl.ds(r0, 8)], kb.at[buf])

    for cl2 in range(4):
        c2 = sc * 4 + cl2

        def zacc(i, carry):
            pltpu.sync_copy(zbuf, acc.at[pl.ds(s * 5120 + i * 128, 128)])
            return carry
        lax.fori_loop(0, 40, zacc, 0)
        plsc.subcore_barrier()

        def fire_gathers(buf):
            for j in range(8):
                pltpu.async_copy(tab_ch.at[c2].at[srcb.at[buf, j]],
                                 vals.at[buf, pl.ds(j * 128, 128)], semg)

        def drain(sem, n):
            for _ in range(n):
                pltpu.make_async_copy(
                    tab_ch.at[c2].at[srcb.at[0, 0]],
                    vals.at[0, pl.ds(0, 128)], sem).wait()

        load_idx(0, 0)
        fire_gathers(0)

        def pair(b, carry):
            for ph in range(2):
                gb = b * 2 + ph
                o = 1 - ph
                load_idx(jnp.minimum(gb + 1, nb - 1), o)
                drain(semg, 8)

                @pl.when(gb > 0)
                def _():
                    drain(sems, 8)
                fire_gathers(o)
                for j in range(8):
                    pltpu.async_copy(vals.at[ph, pl.ds(j * 128, 128)],
                                     acc.at[kb.at[ph, j]], sems, add=True)
            return carry
        lax.fori_loop(0, nb // 2, pair, 0)
        drain(sems, 8)
        drain(semg, 8)
        plsc.subcore_barrier()
        pltpu.sync_copy(acc.at[pl.ds(s * 5000, 5000)],
                        agg.at[pl.ds(s * 5000, 5000), pl.ds(c2 * 16, 16)])
        plsc.subcore_barrier()


@functools.partial(
    pl.kernel,
    out_type=jax.ShapeDtypeStruct((8192, D), jnp.float32),
    mesh=_mesh,
    scratch_types=[
        pltpu.VMEM((2, 128), jnp.int32),
        pltpu.VMEM((256, D), jnp.float32),
        pltpu.SemaphoreType.DMA,
    ],
    compiler_params=pltpu.CompilerParams(use_tc_tiling_on_sc=False),
)
def _pair_kernel(z, pair_rows, out, ib, vals, sem):
    w = lax.axis_index("c") * NS + lax.axis_index("s")
    pltpu.sync_copy(pair_rows.at[pl.ds(w * 2, 2)], ib)
    copies = [
        pltpu.async_copy(z.at[ib.at[r]], vals.at[pl.ds(r * 128, 128)], sem)
        for r in range(2)
    ]
    for cp in copies:
        cp.wait()
    pltpu.sync_copy(vals, out.at[pl.ds(w * 256, 256)])


def _mm_body(x_ref, agg_ref, degp_ref, wf_ref, wr_ref, o_ref, *, relu):
    acc = jnp.dot(x_ref[...], wr_ref[...],
                  preferred_element_type=jnp.float32,
                  precision=lax.Precision.HIGHEST)
    deg = jnp.sum(degp_ref[...], axis=0)
    inv = 1.0 / jnp.maximum(deg, 1.0)
    for r in range(NR):
        acc = acc + jnp.dot(agg_ref[r] * inv[:, r:r + 1], wf_ref[r],
                            preferred_element_type=jnp.float32,
                            precision=lax.Precision.HIGHEST)
    o_ref[...] = jnp.maximum(acc, 0.0) if relu else acc


def _mm_call(xin, agg3, degp3, wf, wr, relu):
    blk = 400
    do = wf.shape[2]
    return pl.pallas_call(
        functools.partial(_mm_body, relu=relu),
        grid=(N // blk,),
        in_specs=[
            pl.BlockSpec((blk, D), lambda i: (i, 0)),
            pl.BlockSpec((NR, blk, D), lambda i: (0, i, 0)),
            pl.BlockSpec((32, blk, NR), lambda i: (0, i, 0)),
            pl.BlockSpec((NR, D, do), lambda i: (0, 0, 0)),
            pl.BlockSpec((D, do), lambda i: (0, 0)),
        ],
        out_specs=pl.BlockSpec((blk, do), lambda i: (i, 0)),
        out_shape=jax.ShapeDtypeStruct((N, do), jnp.float32),
    )(xin, agg3, degp3, wf, wr)


def kernel(x, edge_index, edge_type, indice_pairs, W1, W1_root, W2, W2_root):
    src = edge_index[0]
    dst = edge_index[1]
    pad = E_PAD - E
    src_p = jnp.concatenate([src, jnp.zeros((pad,), jnp.int32)]).reshape(EROWS, 128)
    dst_p = jnp.concatenate([dst, jnp.zeros((pad,), jnp.int32)]).reshape(EROWS, 128)
    ty_p = jnp.concatenate([edge_type, jnp.full((pad,), NR, jnp.int32)]).reshape(EROWS, 128)

    kidx_rows, degp = _deg_kernel(dst_p, ty_p)
    degp3 = degp[:, :NK].reshape(32, N, NR)

    agg1, _ = _agg_kernel(x, src_p, kidx_rows)
    h = _mm_call(x, agg1.reshape(NR, N, D), degp3, W1, W1_root, relu=True)

    agg2, _ = _agg_kernel(h, src_p, kidx_rows)
    w2f = jnp.concatenate([W2, jnp.zeros((NR, D, 78), jnp.float32)], axis=2)
    w2r = jnp.concatenate([W2_root, jnp.zeros((D, 78), jnp.float32)], axis=1)
    z = _mm_call(h, agg2.reshape(NR, N, D), degp3, w2f, w2r, relu=False)

    pair_rows = jnp.concatenate(
        [indice_pairs[:, 0], indice_pairs[:, 1]]).reshape(64, 128)
    zz = _pair_kernel(z, pair_rows)
    return (zz[:4096, :50], zz[4096:, :50])


# speedup vs baseline: 25.3587x; 1.0025x over previous
import functools
import jax
import jax.numpy as jnp
from jax import lax
from jax.experimental import pallas as pl
from jax.experimental.pallas import tpu as pltpu
from jax.experimental.pallas import tpu_sc as plsc

N = 10000
E = 320000
NR = 8
D = 128
NK = NR * N
ACC_ROWS = 81920
E_PAD = 327680
EROWS = E_PAD // 128
ROWS_PER_TILE = EROWS // 16
NC, NS = 2, 16

_mesh = plsc.VectorSubcoreMesh(
    core_axis_name="c", subcore_axis_name="s", num_cores=NC, num_subcores=NS)


def _zero_zbuf(zbuf):
    def zb(i, carry):
        zbuf[i, :] = jnp.zeros((16,), jnp.float32)
        return carry
    lax.fori_loop(0, zbuf.shape[0], zb, 0)


@functools.partial(
    pl.kernel,
    out_type=[
        jax.ShapeDtypeStruct((EROWS, 128), jnp.int32),
        jax.ShapeDtypeStruct((32, 81920), jnp.float32),
    ],
    mesh=_mesh,
    scratch_types=[
        pltpu.VMEM((16, 128), jnp.int32),
        pltpu.VMEM((16, 128), jnp.int32),
        pltpu.VMEM((16, 128), jnp.int32),
        pltpu.VMEM((81920,), jnp.float32),
    ],
    compiler_params=pltpu.CompilerParams(needs_layout_passes=False),
)
def _deg_kernel(dst_rows, ty_rows, kidx_rows, degp, dbuf, tbuf, kbuf, degacc):
    sc = lax.axis_index("c")
    s = lax.axis_index("s")
    w = sc * NS + s

    def zacc(i, carry):
        degacc[pl.ds(i * 16, 16)] = jnp.zeros((16,), jnp.float32)
        return carry
    lax.fori_loop(0, 81920 // 16, zacc, 0)

    ones = jnp.ones((16,), jnp.float32)
    row0 = w * (EROWS // 32)
    for g in range(EROWS // 32 // 16):
        g0 = row0 + g * 16
        pltpu.sync_copy(dst_rows.at[pl.ds(g0, 16)], dbuf)
        pltpu.sync_copy(ty_rows.at[pl.ds(g0, 16)], tbuf)

        def rowfn(r, carry):
            for i in range(8):
                d = dbuf[r, pl.ds(i * 16, 16)]
                t = tbuf[r, pl.ds(i * 16, 16)]
                kbuf[r, pl.ds(i * 16, 16)] = t * N + d
                kd = jnp.where(t >= NR, 80008, d * NR + t)
                plsc.addupdate_scatter(degacc, [kd], ones)
            return carry
        lax.fori_loop(0, 16, rowfn, 0)
        pltpu.sync_copy(kbuf, kidx_rows.at[pl.ds(g0, 16)])
    pltpu.sync_copy(degacc, degp.at[w])


@functools.partial(
    pl.kernel,
    out_type=[
        jax.ShapeDtypeStruct((NK, D), jnp.float32),
        jax.ShapeDtypeStruct((NR, N, 16), jnp.float32),
    ],
    mesh=_mesh,
    scratch_types=[
        pltpu.VMEM((250, 16), jnp.float32),
        pltpu.VMEM((128, 16), jnp.float32),
        pltpu.VMEM((2, 1024), jnp.int32),
        pltpu.VMEM((2, 1024), jnp.int32),
        pltpu.VMEM((2, 1024, 16), jnp.float32),
        pltpu.VMEM_SHARED((ACC_ROWS, 16), jnp.float32),
        pltpu.SemaphoreType.DMA,
        pltpu.SemaphoreType.DMA,
    ],
    compiler_params=pltpu.CompilerParams(use_tc_tiling_on_sc=False),
)
def _agg_kernel(table, src_rows, kidx_rows, agg, tab_ch, relay, zbuf, srcb,
                kb, vals, acc, semg, sems):
    sc = lax.axis_index("c")
    s = lax.axis_index("s")
    nb = ROWS_PER_TILE // 8
    cl = s // 4
    q = s % 4
    c = sc * 4 + cl
    for i in range(10):
        r0 = q * 2500 + i * 250
        pltpu.sync_copy(table.at[pl.ds(r0, 250), pl.ds(c * 16, 16)], relay)
        pltpu.sync_copy(relay, tab_ch.at[c, pl.ds(r0, 250), :])
    _zero_zbuf(zbuf)
    plsc.subcore_barrier()

    def load_idx(gb, buf):
        r0 = s * (ROWS_PER_TILE // 8) + gb
        pltpu.sync_copy(src_rows.at[r0], srcb.at[buf])
        pltpu.sync_copy(kidx_rows.at[r0], kb.at[buf])

    for cl2 in range(4):
        c2 = sc * 4 + cl2

        def zacc(i, carry):
            pltpu.sync_copy(zbuf, acc.at[pl.ds(s * 5120 + i * 128, 128)])
            return carry
        lax.fori_loop(0, 40, zacc, 0)
        plsc.subcore_barrier()

        def fire_gathers(buf):
            pltpu.async_copy(tab_ch.at[c2].at[srcb.at[buf]], vals.at[buf],
                             semg)

        def drain(sem):
            pltpu.make_async_copy(tab_ch.at[c2].at[srcb.at[0]], vals.at[0],
                                  sem).wait()

        load_idx(0, 0)
        fire_gathers(0)

        def pair(b, carry):
            for ph in range(2):
                gb = b * 2 + ph
                o = 1 - ph
                load_idx(jnp.minimum(gb + 1, nb - 1), o)
                drain(semg)

                @pl.when(gb > 0)
                def _():
                    drain(sems)
                fire_gathers(o)
                pltpu.async_copy(vals.at[ph], acc.at[kb.at[ph]], sems,
                                 add=True)
            return carry
        lax.fori_loop(0, nb // 2, pair, 0)
        drain(sems)
        drain(semg)
        plsc.subcore_barrier()
        pltpu.sync_copy(acc.at[pl.ds(s * 5000, 5000)],
                        agg.at[pl.ds(s * 5000, 5000), pl.ds(c2 * 16, 16)])
        plsc.subcore_barrier()


@functools.partial(
    pl.kernel,
    out_type=jax.ShapeDtypeStruct((8192, D), jnp.float32),
    mesh=_mesh,
    scratch_types=[
        pltpu.VMEM((2, 128), jnp.int32),
        pltpu.VMEM((256, D), jnp.float32),
        pltpu.SemaphoreType.DMA,
    ],
    compiler_params=pltpu.CompilerParams(use_tc_tiling_on_sc=False),
)
def _pair_kernel(z, pair_rows, out, ib, vals, sem):
    w = lax.axis_index("c") * NS + lax.axis_index("s")
    pltpu.sync_copy(pair_rows.at[pl.ds(w * 2, 2)], ib)
    copies = [
        pltpu.async_copy(z.at[ib.at[r]], vals.at[pl.ds(r * 128, 128)], sem)
        for r in range(2)
    ]
    for cp in copies:
        cp.wait()
    pltpu.sync_copy(vals, out.at[pl.ds(w * 256, 256)])


def _mm_body(x_ref, agg_ref, degp_ref, wf_ref, wr_ref, o_ref, *, relu):
    acc = jnp.dot(x_ref[...], wr_ref[...],
                  preferred_element_type=jnp.float32,
                  precision=lax.Precision.HIGHEST)
    deg = jnp.sum(degp_ref[...], axis=0)
    inv = 1.0 / jnp.maximum(deg, 1.0)
    for r in range(NR):
        acc = acc + jnp.dot(agg_ref[r] * inv[:, r:r + 1], wf_ref[r],
                            preferred_element_type=jnp.float32,
                            precision=lax.Precision.HIGHEST)
    o_ref[...] = jnp.maximum(acc, 0.0) if relu else acc


def _mm_call(xin, agg3, degp3, wf, wr, relu):
    blk = 400
    do = wf.shape[2]
    return pl.pallas_call(
        functools.partial(_mm_body, relu=relu),
        grid=(N // blk,),
        in_specs=[
            pl.BlockSpec((blk, D), lambda i: (i, 0)),
            pl.BlockSpec((NR, blk, D), lambda i: (0, i, 0)),
            pl.BlockSpec((32, blk, NR), lambda i: (0, i, 0)),
            pl.BlockSpec((NR, D, do), lambda i: (0, 0, 0)),
            pl.BlockSpec((D, do), lambda i: (0, 0)),
        ],
        out_specs=pl.BlockSpec((blk, do), lambda i: (i, 0)),
        out_shape=jax.ShapeDtypeStruct((N, do), jnp.float32),
    )(xin, agg3, degp3, wf, wr)


def kernel(x, edge_index, edge_type, indice_pairs, W1, W1_root, W2, W2_root):
    src = edge_index[0]
    dst = edge_index[1]
    pad = E_PAD - E
    src_p = jnp.concatenate([src, jnp.zeros((pad,), jnp.int32)]).reshape(320, 1024)
    dst_p = jnp.concatenate([dst, jnp.zeros((pad,), jnp.int32)]).reshape(EROWS, 128)
    ty_p = jnp.concatenate([edge_type, jnp.full((pad,), NR, jnp.int32)]).reshape(EROWS, 128)

    kidx_rows, degp = _deg_kernel(dst_p, ty_p)
    kidx_b = kidx_rows.reshape(320, 1024)
    degp3 = degp[:, :NK].reshape(32, N, NR)

    agg1, _ = _agg_kernel(x, src_p, kidx_b)
    h = _mm_call(x, agg1.reshape(NR, N, D), degp3, W1, W1_root, relu=True)

    agg2, _ = _agg_kernel(h, src_p, kidx_b)
    w2f = jnp.concatenate([W2, jnp.zeros((NR, D, 78), jnp.float32)], axis=2)
    w2r = jnp.concatenate([W2_root, jnp.zeros((D, 78), jnp.float32)], axis=1)
    z = _mm_call(h, agg2.reshape(NR, N, D), degp3, w2f, w2r, relu=False)

    pair_rows = jnp.concatenate(
        [indice_pairs[:, 0], indice_pairs[:, 1]]).reshape(64, 128)
    zz = _pair_kernel(z, pair_rows)
    return (zz[:4096, :50], zz[4096:, :50])


# speedup vs baseline: 33.4872x; 1.3205x over previous
import functools
import jax
import jax.numpy as jnp
import numpy as np
from jax import lax
from jax.experimental import pallas as pl
from jax.experimental.pallas import tpu as pltpu
from jax.experimental.pallas import tpu_sc as plsc

N = 10000
E = 320000
NR = 8
D = 128
NK = NR * N
ACC_ROWS = 81920
E_PAD = 327680
EROWS = E_PAD // 128
ROWS_PER_TILE = EROWS // 16
NC, NS = 2, 16

_mesh = plsc.VectorSubcoreMesh(
    core_axis_name="c", subcore_axis_name="s", num_cores=NC, num_subcores=NS)

_PERM = np.zeros((D,), dtype=np.int32)
for _dc in range(4):
    for _i in range(16):
        _PERM[_dc * 32 + 2 * _i] = _dc * 32 + _i
        _PERM[_dc * 32 + 2 * _i + 1] = _dc * 32 + 16 + _i


def _zero_zbuf(zbuf):
    def zb(i, carry):
        zbuf[i, :] = jnp.zeros((16,), jnp.float32)
        return carry
    lax.fori_loop(0, zbuf.shape[0], zb, 0)


@functools.partial(
    pl.kernel,
    out_type=[
        jax.ShapeDtypeStruct((EROWS, 128), jnp.int32),
        jax.ShapeDtypeStruct((32, 81920), jnp.float32),
    ],
    mesh=_mesh,
    scratch_types=[
        pltpu.VMEM((16, 128), jnp.int32),
        pltpu.VMEM((16, 128), jnp.int32),
        pltpu.VMEM((16, 128), jnp.int32),
        pltpu.VMEM((81920,), jnp.float32),
    ],
    compiler_params=pltpu.CompilerParams(needs_layout_passes=False),
)
def _deg_kernel(dst_rows, ty_rows, kidx_rows, degp, dbuf, tbuf, kbuf, degacc):
    sc = lax.axis_index("c")
    s = lax.axis_index("s")
    w = sc * NS + s

    def zacc(i, carry):
        degacc[pl.ds(i * 16, 16)] = jnp.zeros((16,), jnp.float32)
        return carry
    lax.fori_loop(0, 81920 // 16, zacc, 0)

    ones = jnp.ones((16,), jnp.float32)
    row0 = w * (EROWS // 32)
    for g in range(EROWS // 32 // 16):
        g0 = row0 + g * 16
        pltpu.sync_copy(dst_rows.at[pl.ds(g0, 16)], dbuf)
        pltpu.sync_copy(ty_rows.at[pl.ds(g0, 16)], tbuf)

        def rowfn(r, carry):
            for i in range(8):
                d = dbuf[r, pl.ds(i * 16, 16)]
                t = tbuf[r, pl.ds(i * 16, 16)]
                kbuf[r, pl.ds(i * 16, 16)] = t * N + d
                kd = jnp.where(t >= NR, 80008, d * NR + t)
                plsc.addupdate_scatter(degacc, [kd], ones)
            return carry
        lax.fori_loop(0, 16, rowfn, 0)
        pltpu.sync_copy(kbuf, kidx_rows.at[pl.ds(g0, 16)])
    pltpu.sync_copy(degacc, degp.at[w])


@functools.partial(
    pl.kernel,
    out_type=[
        jax.ShapeDtypeStruct((NK, D), jnp.bfloat16),
        jax.ShapeDtypeStruct((4, N, 32), jnp.bfloat16),
    ],
    mesh=_mesh,
    scratch_types=[
        pltpu.VMEM((125, 32), jnp.float32),
        pltpu.VMEM((125, 32), jnp.bfloat16),
        pltpu.VMEM((128, 32), jnp.bfloat16),
        pltpu.VMEM((2, 1024), jnp.int32),
        pltpu.VMEM((2, 1024), jnp.int32),
        pltpu.VMEM((2, 1024, 32), jnp.bfloat16),
        pltpu.VMEM_SHARED((ACC_ROWS, 32), jnp.bfloat16),
        pltpu.SemaphoreType.DMA,
        pltpu.SemaphoreType.DMA,
    ],
    compiler_params=pltpu.CompilerParams(
        use_tc_tiling_on_sc=False, needs_layout_passes=False),
)
def _agg_kernel(table, src_rows, kidx_rows, agg, tab_ch, xbuf, pkbuf, zbuf,
                srcb, kb, vals, acc, semg, sems):
    sc = lax.axis_index("c")
    s = lax.axis_index("s")
    nb = ROWS_PER_TILE // 8
    dcl = s // 8
    sub = s % 8
    dc = sc * 2 + dcl
    for p in range(10):
        r0 = sub * 1250 + p * 125
        pltpu.sync_copy(table.at[pl.ds(r0, 125), pl.ds(dc * 32, 32)], xbuf)

        def packrow(r, carry):
            a = xbuf[r, pl.ds(0, 16)]
            b = xbuf[r, pl.ds(16, 16)]
            pkbuf[r, :] = plsc.pack(a, b, format=plsc.PackFormat.INTERLEAVED)
            return carry
        lax.fori_loop(0, 125, packrow, 0)
        pltpu.sync_copy(pkbuf, tab_ch.at[dc, pl.ds(r0, 125), :])

    def zb(i, carry):
        zbuf[i, :] = jnp.zeros((32,), jnp.bfloat16)
        return carry
    lax.fori_loop(0, 128, zb, 0)
    plsc.subcore_barrier()

    def load_idx(gb, buf):
        r0 = s * nb + gb
        pltpu.sync_copy(src_rows.at[r0], srcb.at[buf])
        pltpu.sync_copy(kidx_rows.at[r0], kb.at[buf])

    for dcl2 in range(2):
        dc2 = sc * 2 + dcl2

        def zacc(i, carry):
            pltpu.sync_copy(zbuf, acc.at[pl.ds(s * 5120 + i * 128, 128)])
            return carry
        lax.fori_loop(0, 40, zacc, 0)
        plsc.subcore_barrier()

        def fire_gathers(buf):
            pltpu.async_copy(tab_ch.at[dc2].at[srcb.at[buf]], vals.at[buf],
                             semg)

        def drain(sem):
            pltpu.make_async_copy(tab_ch.at[dc2].at[srcb.at[0]], vals.at[0],
                                  sem).wait()

        load_idx(0, 0)
        fire_gathers(0)

        def pair(b, carry):
            for ph in range(2):
                gb = b * 2 + ph
                o = 1 - ph
                load_idx(jnp.minimum(gb + 1, nb - 1), o)
                drain(semg)

                @pl.when(gb > 0)
                def _():
                    drain(sems)
                fire_gathers(o)
                pltpu.async_copy(vals.at[ph], acc.at[kb.at[ph]], sems,
                                 add=True)
            return carry
        lax.fori_loop(0, nb // 2, pair, 0)
        drain(sems)
        drain(semg)
        plsc.subcore_barrier()
        pltpu.sync_copy(acc.at[pl.ds(s * 5000, 5000)],
                        agg.at[pl.ds(s * 5000, 5000), pl.ds(dc2 * 32, 32)])
        plsc.subcore_barrier()


@functools.partial(
    pl.kernel,
    out_type=jax.ShapeDtypeStruct((8192, D), jnp.float32),
    mesh=_mesh,
    scratch_types=[
        pltpu.VMEM((2, 128), jnp.int32),
        pltpu.VMEM((256, D), jnp.float32),
        pltpu.SemaphoreType.DMA,
    ],
    compiler_params=pltpu.CompilerParams(use_tc_tiling_on_sc=False),
)
def _pair_kernel(z, pair_rows, out, ib, vals, sem):
    w = lax.axis_index("c") * NS + lax.axis_index("s")
    pltpu.sync_copy(pair_rows.at[pl.ds(w * 2, 2)], ib)
    copies = [
        pltpu.async_copy(z.at[ib.at[r]], vals.at[pl.ds(r * 128, 128)], sem)
        for r in range(2)
    ]
    for cp in copies:
        cp.wait()
    pltpu.sync_copy(vals, out.at[pl.ds(w * 256, 256)])


def _mm_body(x_ref, agg_ref, degp_ref, wf_ref, wr_ref, o_ref, *, relu):
    acc = jnp.dot(x_ref[...], wr_ref[...],
                  preferred_element_type=jnp.float32,
                  precision=lax.Precision.HIGHEST)
    deg = jnp.sum(degp_ref[...], axis=0)
    inv = 1.0 / jnp.maximum(deg, 1.0)
    for r in range(NR):
        acc = acc + inv[:, r:r + 1] * jnp.dot(
            agg_ref[r], wf_ref[r], preferred_element_type=jnp.float32)
    o_ref[...] = jnp.maximum(acc, 0.0) if relu else acc


def _mm_call(xin, agg3, degp3, wf, wr, relu):
    blk = 400
    do = wf.shape[2]
    return pl.pallas_call(
        functools.partial(_mm_body, relu=relu),
        grid=(N // blk,),
        in_specs=[
            pl.BlockSpec((blk, D), lambda i: (i, 0)),
            pl.BlockSpec((NR, blk, D), lambda i: (0, i, 0)),
            pl.BlockSpec((32, blk, NR), lambda i: (0, i, 0)),
            pl.BlockSpec((NR, D, do), lambda i: (0, 0, 0)),
            pl.BlockSpec((D, do), lambda i: (0, 0)),
        ],
        out_specs=pl.BlockSpec((blk, do), lambda i: (i, 0)),
        out_shape=jax.ShapeDtypeStruct((N, do), jnp.float32),
    )(xin, agg3, degp3, wf, wr)


def kernel(x, edge_index, edge_type, indice_pairs, W1, W1_root, W2, W2_root):
    src = edge_index[0]
    dst = edge_index[1]
    pad = E_PAD - E
    src_p = jnp.concatenate([src, jnp.zeros((pad,), jnp.int32)]).reshape(320, 1024)
    dst_p = jnp.concatenate([dst, jnp.zeros((pad,), jnp.int32)]).reshape(EROWS, 128)
    ty_p = jnp.concatenate([edge_type, jnp.full((pad,), NR, jnp.int32)]).reshape(EROWS, 128)

    kidx_rows, degp = _deg_kernel(dst_p, ty_p)
    kidx_b = kidx_rows.reshape(320, 1024)
    degp3 = degp[:, :NK].reshape(32, N, NR)

    w1f = W1[:, _PERM, :]
    agg1, _ = _agg_kernel(x, src_p, kidx_b)
    h = _mm_call(x, agg1.reshape(NR, N, D), degp3, w1f, W1_root, relu=True)

    agg2, _ = _agg_kernel(h, src_p, kidx_b)
    w2f = jnp.concatenate(
        [W2[:, _PERM, :], jnp.zeros((NR, D, 78), jnp.float32)], axis=2)
    w2r = jnp.concatenate([W2_root, jnp.zeros((D, 78), jnp.float32)], axis=1)
    z = _mm_call(h, agg2.reshape(NR, N, D), degp3, w2f, w2r, relu=False)

    pair_rows = jnp.concatenate(
        [indice_pairs[:, 0], indice_pairs[:, 1]]).reshape(64, 128)
    zz = _pair_kernel(z, pair_rows)
    return (zz[:4096, :50], zz[4096:, :50])


# speedup vs baseline: 33.9792x; 1.0147x over previous
import functools
import jax
import jax.numpy as jnp
import numpy as np
from jax import lax
from jax.experimental import pallas as pl
from jax.experimental.pallas import tpu as pltpu
from jax.experimental.pallas import tpu_sc as plsc

N = 10000
E = 320000
NR = 8
D = 128
NK = NR * N
ACC_ROWS = 81920
E_PAD = 327680
EROWS = E_PAD // 128
ROWS_PER_TILE = EROWS // 16
NC, NS = 2, 16

_mesh = plsc.VectorSubcoreMesh(
    core_axis_name="c", subcore_axis_name="s", num_cores=NC, num_subcores=NS)

_PERM = np.zeros((D,), dtype=np.int32)
for _dc in range(4):
    for _i in range(16):
        _PERM[_dc * 32 + 2 * _i] = _dc * 32 + _i
        _PERM[_dc * 32 + 2 * _i + 1] = _dc * 32 + 16 + _i


def _zero_zbuf(zbuf):
    def zb(i, carry):
        zbuf[i, :] = jnp.zeros((16,), jnp.float32)
        return carry
    lax.fori_loop(0, zbuf.shape[0], zb, 0)


@functools.partial(
    pl.kernel,
    out_type=[
        jax.ShapeDtypeStruct((EROWS, 128), jnp.int32),
        jax.ShapeDtypeStruct((32, 81920), jnp.float32),
    ],
    mesh=_mesh,
    scratch_types=[
        pltpu.VMEM((16, 128), jnp.int32),
        pltpu.VMEM((16, 128), jnp.int32),
        pltpu.VMEM((16, 128), jnp.int32),
        pltpu.VMEM((81920,), jnp.float32),
    ],
    compiler_params=pltpu.CompilerParams(needs_layout_passes=False),
)
def _deg_kernel(dst_rows, ty_rows, zeros_in, kidx_rows, degp, dbuf, tbuf,
                kbuf, degacc):
    sc = lax.axis_index("c")
    s = lax.axis_index("s")
    w = sc * NS + s
    pltpu.sync_copy(zeros_in, degacc)

    ones = jnp.ones((16,), jnp.float32)
    row0 = w * (EROWS // 32)
    for g in range(EROWS // 32 // 16):
        g0 = row0 + g * 16
        pltpu.sync_copy(dst_rows.at[pl.ds(g0, 16)], dbuf)
        pltpu.sync_copy(ty_rows.at[pl.ds(g0, 16)], tbuf)

        def rowfn(r, carry):
            for i in range(8):
                d = dbuf[r, pl.ds(i * 16, 16)]
                t = tbuf[r, pl.ds(i * 16, 16)]
                kbuf[r, pl.ds(i * 16, 16)] = t * N + d
                kd = jnp.where(t >= NR, 80008, d * NR + t)
                plsc.addupdate_scatter(degacc, [kd], ones)
            return carry
        lax.fori_loop(0, 16, rowfn, 0)
        pltpu.sync_copy(kbuf, kidx_rows.at[pl.ds(g0, 16)])
    pltpu.sync_copy(degacc, degp.at[w])


@functools.partial(
    pl.kernel,
    out_type=[
        jax.ShapeDtypeStruct((NK, D), jnp.bfloat16),
        jax.ShapeDtypeStruct((4, N, 32), jnp.bfloat16),
    ],
    mesh=_mesh,
    scratch_types=[
        pltpu.VMEM((125, 32), jnp.float32),
        pltpu.VMEM((125, 32), jnp.bfloat16),
        pltpu.VMEM((128, 32), jnp.bfloat16),
        pltpu.VMEM((2, 1024), jnp.int32),
        pltpu.VMEM((2, 1024), jnp.int32),
        pltpu.VMEM((2, 1024, 32), jnp.bfloat16),
        pltpu.VMEM_SHARED((ACC_ROWS, 32), jnp.bfloat16),
        pltpu.SemaphoreType.DMA,
        pltpu.SemaphoreType.DMA,
    ],
    compiler_params=pltpu.CompilerParams(
        use_tc_tiling_on_sc=False, needs_layout_passes=False),
)
def _agg_kernel(table, src_rows, kidx_rows, agg, tab_ch, xbuf, pkbuf, zbuf,
                srcb, kb, vals, acc, semg, sems):
    sc = lax.axis_index("c")
    s = lax.axis_index("s")
    nb = ROWS_PER_TILE // 8
    dcl = s // 8
    sub = s % 8
    dc = sc * 2 + dcl
    for p in range(10):
        r0 = sub * 1250 + p * 125
        pltpu.sync_copy(table.at[pl.ds(r0, 125), pl.ds(dc * 32, 32)], xbuf)

        def packrow(r, carry):
            a = xbuf[r, pl.ds(0, 16)]
            b = xbuf[r, pl.ds(16, 16)]
            pkbuf[r, :] = plsc.pack(a, b, format=plsc.PackFormat.INTERLEAVED)
            return carry
        lax.fori_loop(0, 125, packrow, 0)
        pltpu.sync_copy(pkbuf, tab_ch.at[dc, pl.ds(r0, 125), :])

    def zb(i, carry):
        zbuf[i, :] = jnp.zeros((32,), jnp.bfloat16)
        return carry
    lax.fori_loop(0, 128, zb, 0)
    plsc.subcore_barrier()

    def load_idx(gb, buf):
        r0 = s * nb + gb
        pltpu.sync_copy(src_rows.at[r0], srcb.at[buf])
        pltpu.sync_copy(kidx_rows.at[r0], kb.at[buf])

    for dcl2 in range(2):
        dc2 = sc * 2 + dcl2

        def zacc(i, carry):
            pltpu.sync_copy(zbuf, acc.at[pl.ds(s * 5120 + i * 128, 128)])
            return carry
        lax.fori_loop(0, 40, zacc, 0)
        plsc.subcore_barrier()

        def fire_gathers(buf):
            pltpu.async_copy(tab_ch.at[dc2].at[srcb.at[buf]], vals.at[buf],
                             semg)

        def drain(sem):
            pltpu.make_async_copy(tab_ch.at[dc2].at[srcb.at[0]], vals.at[0],
                                  sem).wait()

        load_idx(0, 0)
        fire_gathers(0)

        def pair(b, carry):
            for ph in range(2):
                gb = b * 2 + ph
                o = 1 - ph
                load_idx(jnp.minimum(gb + 1, nb - 1), o)
                drain(semg)

                @pl.when(gb > 0)
                def _():
                    drain(sems)
                fire_gathers(o)
                pltpu.async_copy(vals.at[ph], acc.at[kb.at[ph]], sems,
                                 add=True)
            return carry
        lax.fori_loop(0, nb // 2, pair, 0)
        drain(sems)
        drain(semg)
        plsc.subcore_barrier()
        pltpu.sync_copy(acc.at[pl.ds(s * 5000, 5000)],
                        agg.at[pl.ds(s * 5000, 5000), pl.ds(dc2 * 32, 32)])
        plsc.subcore_barrier()


@functools.partial(
    pl.kernel,
    out_type=jax.ShapeDtypeStruct((8192, D), jnp.float32),
    mesh=_mesh,
    scratch_types=[
        pltpu.VMEM((2, 128), jnp.int32),
        pltpu.VMEM((256, D), jnp.float32),
        pltpu.SemaphoreType.DMA,
    ],
    compiler_params=pltpu.CompilerParams(use_tc_tiling_on_sc=False),
)
def _pair_kernel(z, pair_rows, out, ib, vals, sem):
    w = lax.axis_index("c") * NS + lax.axis_index("s")
    pltpu.sync_copy(pair_rows.at[pl.ds(w * 2, 2)], ib)
    copies = [
        pltpu.async_copy(z.at[ib.at[r]], vals.at[pl.ds(r * 128, 128)], sem)
        for r in range(2)
    ]
    for cp in copies:
        cp.wait()
    pltpu.sync_copy(vals, out.at[pl.ds(w * 256, 256)])


def _mm_body(x_ref, agg_ref, degp_ref, wf_ref, wr_ref, o_ref, *, relu):
    acc = jnp.dot(x_ref[...], wr_ref[...],
                  preferred_element_type=jnp.float32)
    deg = jnp.sum(degp_ref[...], axis=0)
    inv = 1.0 / jnp.maximum(deg, 1.0)
    for r in range(NR):
        acc = acc + inv[:, r:r + 1] * jnp.dot(
            agg_ref[r], wf_ref[r], preferred_element_type=jnp.float32)
    o_ref[...] = jnp.maximum(acc, 0.0) if relu else acc


def _mm_call(xin, agg3, degp3, wf, wr, relu):
    blk = 400
    do = wf.shape[2]
    return pl.pallas_call(
        functools.partial(_mm_body, relu=relu),
        grid=(N // blk,),
        in_specs=[
            pl.BlockSpec((blk, D), lambda i: (i, 0)),
            pl.BlockSpec((NR, blk, D), lambda i: (0, i, 0)),
            pl.BlockSpec((32, blk, NR), lambda i: (0, i, 0)),
            pl.BlockSpec((NR, D, do), lambda i: (0, 0, 0)),
            pl.BlockSpec((D, do), lambda i: (0, 0)),
        ],
        out_specs=pl.BlockSpec((blk, do), lambda i: (i, 0)),
        out_shape=jax.ShapeDtypeStruct((N, do), jnp.float32),
    )(xin, agg3, degp3, wf, wr)


def kernel(x, edge_index, edge_type, indice_pairs, W1, W1_root, W2, W2_root):
    src = edge_index[0]
    dst = edge_index[1]
    pad = E_PAD - E
    src_p = jnp.concatenate([src, jnp.zeros((pad,), jnp.int32)]).reshape(320, 1024)
    dst_p = jnp.concatenate([dst, jnp.zeros((pad,), jnp.int32)]).reshape(EROWS, 128)
    ty_p = jnp.concatenate([edge_type, jnp.full((pad,), NR, jnp.int32)]).reshape(EROWS, 128)

    kidx_rows, degp = _deg_kernel(dst_p, ty_p, jnp.zeros((81920,), jnp.float32))
    kidx_b = kidx_rows.reshape(320, 1024)
    degp3 = degp[:, :NK].reshape(32, N, NR)

    w1f = W1[:, _PERM, :]
    agg1, _ = _agg_kernel(x, src_p, kidx_b)
    h = _mm_call(x, agg1.reshape(NR, N, D), degp3, w1f, W1_root, relu=True)

    agg2, _ = _agg_kernel(h, src_p, kidx_b)
    w2f = jnp.concatenate(
        [W2[:, _PERM, :], jnp.zeros((NR, D, 78), jnp.float32)], axis=2)
    w2r = jnp.concatenate([W2_root, jnp.zeros((D, 78), jnp.float32)], axis=1)
    z = _mm_call(h, agg2.reshape(NR, N, D), degp3, w2f, w2r, relu=False)

    pair_rows = jnp.concatenate(
        [indice_pairs[:, 0], indice_pairs[:, 1]]).reshape(64, 128)
    zz = _pair_kernel(z, pair_rows)
    return (zz[:4096, :50], zz[4096:, :50])
